# Initial kernel scaffold; baseline (speedup 1.0000x reference)
import jax
import jax.numpy as jnp
from jax.experimental import pallas as pl


def kernel(node_emb, edge_emb, neighbour_list, e_W1, e_b1, e_W2, e_b2, e_g, e_beta, n_W1, n_b1, n_W2, n_b2, n_g, n_beta, r_W, r_b, s_W, s_b):
    raise NotImplementedError("write your pallas kernel here")


# speedup vs baseline: 2.3322x; 2.3322x over previous
import functools

import jax
import jax.numpy as jnp
from jax import lax
from jax.experimental import pallas as pl
from jax.experimental.pallas import tpu as pltpu
from jax.experimental.pallas import tpu_sc as plsc

N = 10000
E = 320000
C = 128
H = 128

N_PAD = 10240
NW = 32
EPW = E // NW
KG = 80
EPT = E // 16
KS = 80
ROWS_PT = N_PAD // 16
ZROWS = ROWS_PT // 8


@functools.cache
def _sc_kernels():
    mesh = plsc.VectorSubcoreMesh(core_axis_name="c", subcore_axis_name="s")

    @functools.partial(
        pl.kernel,
        out_type=[
            jax.ShapeDtypeStruct((E, C), jnp.float32),
            jax.ShapeDtypeStruct((E, C), jnp.float32),
        ],
        mesh=mesh,
        scratch_types=[
            pltpu.VMEM((KG,), jnp.int32),
            pltpu.VMEM((KG,), jnp.int32),
            pltpu.VMEM((KG, C), jnp.float32),
            pltpu.VMEM((KG, C), jnp.float32),
            pltpu.SemaphoreType.DMA,
            pltpu.SemaphoreType.DMA,
        ],
    )
    def sc_gather(node_hbm, snd_hbm, rcv_hbm, outs_hbm, outr_hbm,
                  sidx, ridx, srows, rrows, sem_s, sem_r):
        wid = lax.axis_index("s") * 2 + lax.axis_index("c")
        base = wid * EPW

        def body(ci, carry):
            off = base + ci * KG
            pltpu.sync_copy(snd_hbm.at[pl.ds(off, KG)], sidx)
            pltpu.sync_copy(rcv_hbm.at[pl.ds(off, KG)], ridx)
            cs = pltpu.async_copy(node_hbm.at[sidx], srows, sem_s)
            cr = pltpu.async_copy(node_hbm.at[ridx], rrows, sem_r)
            cs.wait()
            cr.wait()
            pltpu.sync_copy(srows, outs_hbm.at[pl.ds(off, KG)])
            pltpu.sync_copy(rrows, outr_hbm.at[pl.ds(off, KG)])
            return carry

        lax.fori_loop(0, EPW // KG, body, 0)

    @functools.partial(
        pl.kernel,
        out_type=jax.ShapeDtypeStruct((2 * N_PAD, C), jnp.float32),
        mesh=mesh,
        scratch_types=[
            pltpu.VMEM_SHARED((N_PAD, C), jnp.float32),
            pltpu.VMEM((KS,), jnp.int32),
            pltpu.VMEM((KS, C), jnp.float32),
            pltpu.VMEM((ZROWS, C), jnp.float32),
        ],
    )
    def sc_scatter(ws_hbm, wr_hbm, snd_hbm, rcv_hbm, out_hbm,
                   acc, idxb, rowsb, zb):
        cid = lax.axis_index("c")
        sid = lax.axis_index("s")

        def zrow(i, carry):
            for j in range(C // 16):
                zb[i, pl.ds(j * 16, 16)] = jnp.zeros((16,), jnp.float32)
            return carry

        lax.fori_loop(0, ZROWS, zrow, 0)
        for i in range(ROWS_PT // ZROWS):
            pltpu.sync_copy(
                zb, acc.at[pl.ds(sid * ROWS_PT + i * ZROWS, ZROWS)])
        plsc.subcore_barrier()

        def process(data_hbm, idx_hbm):
            def body(ci, carry):
                off = sid * EPT + ci * KS
                pltpu.sync_copy(idx_hbm.at[pl.ds(off, KS)], idxb)
                pltpu.sync_copy(data_hbm.at[pl.ds(off, KS)], rowsb)
                pltpu.sync_copy(rowsb, acc.at[idxb], add=True)
                return carry

            lax.fori_loop(0, EPT // KS, body, 0)

        @pl.when(cid == 0)
        def _():
            process(ws_hbm, snd_hbm)

        @pl.when(cid == 1)
        def _():
            process(wr_hbm, rcv_hbm)

        plsc.subcore_barrier()
        pltpu.sync_copy(
            acc.at[pl.ds(sid * ROWS_PT, ROWS_PT)],
            out_hbm.at[pl.ds(cid * N_PAD + sid * ROWS_PT, ROWS_PT)])

    return sc_gather, sc_scatter


def _edge_body(edge_ref, s_ref, r_ref, w1a_ref, w1b_ref, w1c_ref, b1_ref,
               w2_ref, b2_ref, g_ref, beta_ref, rw_ref, rb_ref, sw_ref,
               sb_ref, eout_ref, ws_ref, wr_ref):
    edge = edge_ref[...]
    h = (jnp.dot(edge, w1a_ref[...], preferred_element_type=jnp.float32)
         + jnp.dot(s_ref[...], w1b_ref[...], preferred_element_type=jnp.float32)
         + jnp.dot(r_ref[...], w1c_ref[...], preferred_element_type=jnp.float32)
         + b1_ref[...])
    h = h * jax.nn.sigmoid(h)
    m = jnp.dot(h, w2_ref[...], preferred_element_type=jnp.float32) + b2_ref[...]
    mu = jnp.mean(m, axis=-1, keepdims=True)
    var = jnp.mean((m - mu) * (m - mu), axis=-1, keepdims=True)
    nef = (m - mu) * lax.rsqrt(var + 1e-5) * g_ref[...] + beta_ref[...]
    ra = jax.nn.sigmoid(
        jnp.sum(edge * rw_ref[...], axis=-1, keepdims=True) + rb_ref[0, 0])
    sa = jax.nn.sigmoid(
        jnp.sum(edge * sw_ref[...], axis=-1, keepdims=True) + sb_ref[0, 0])
    eout_ref[...] = edge + nef
    ws_ref[...] = nef * sa
    wr_ref[...] = nef * ra


def _tc_edge(edge_emb, s_rows, r_rows, e_W1, e_b1, e_W2, e_b2, e_g, e_beta,
             r_W, r_b, s_W, s_b):
    B = 512
    grid = (E // B,)
    row = lambda i: (i, 0)
    rep = lambda i: (0, 0)
    blk = pl.BlockSpec((B, C), row)
    wspec = pl.BlockSpec((C, H), rep)
    vspec = pl.BlockSpec((1, C), rep)
    sspec = pl.BlockSpec((1, 1), rep)
    return pl.pallas_call(
        _edge_body,
        grid=grid,
        in_specs=[blk, blk, blk, wspec, wspec, wspec, vspec,
                  pl.BlockSpec((H, C), rep), vspec, vspec, vspec,
                  vspec, sspec, vspec, sspec],
        out_specs=[blk, blk, blk],
        out_shape=[jax.ShapeDtypeStruct((E, C), jnp.float32)] * 3,
    )(edge_emb, s_rows, r_rows,
      e_W1[0:C], e_W1[C:2 * C], e_W1[2 * C:3 * C], e_b1.reshape(1, H),
      e_W2, e_b2.reshape(1, C), e_g.reshape(1, C), e_beta.reshape(1, C),
      r_W.reshape(1, C), r_b.reshape(1, 1),
      s_W.reshape(1, C), s_b.reshape(1, 1))


def _node_body(node_ref, snt_ref, rcv_ref, w1a_ref, w1b_ref, w1c_ref, b1_ref,
               w2_ref, b2_ref, g_ref, beta_ref, out_ref):
    node = node_ref[...]
    h = (jnp.dot(node, w1a_ref[...], preferred_element_type=jnp.float32)
         + jnp.dot(snt_ref[...], w1b_ref[...], preferred_element_type=jnp.float32)
         + jnp.dot(rcv_ref[...], w1c_ref[...], preferred_element_type=jnp.float32)
         + b1_ref[...])
    h = h * jax.nn.sigmoid(h)
    m = jnp.dot(h, w2_ref[...], preferred_element_type=jnp.float32) + b2_ref[...]
    mu = jnp.mean(m, axis=-1, keepdims=True)
    var = jnp.mean((m - mu) * (m - mu), axis=-1, keepdims=True)
    nnf = (m - mu) * lax.rsqrt(var + 1e-5) * g_ref[...] + beta_ref[...]
    out_ref[...] = node + nnf


def _tc_node(node_pad, sent_pad, recv_pad, n_W1, n_b1, n_W2, n_b2, n_g,
             n_beta):
    B = 512
    grid = (N_PAD // B,)
    row = lambda i: (i, 0)
    rep = lambda i: (0, 0)
    blk = pl.BlockSpec((B, C), row)
    wspec = pl.BlockSpec((C, H), rep)
    vspec = pl.BlockSpec((1, C), rep)
    return pl.pallas_call(
        _node_body,
        grid=grid,
        in_specs=[blk, blk, blk, wspec, wspec, wspec, vspec,
                  pl.BlockSpec((H, C), rep), vspec, vspec, vspec],
        out_specs=blk,
        out_shape=jax.ShapeDtypeStruct((N_PAD, C), jnp.float32),
    )(node_pad, sent_pad, recv_pad,
      n_W1[0:C], n_W1[C:2 * C], n_W1[2 * C:3 * C], n_b1.reshape(1, H),
      n_W2, n_b2.reshape(1, C), n_g.reshape(1, C), n_beta.reshape(1, C))


def kernel(node_emb, edge_emb, neighbour_list, e_W1, e_b1, e_W2, e_b2, e_g,
           e_beta, n_W1, n_b1, n_W2, n_b2, n_g, n_beta, r_W, r_b, s_W, s_b):
    senders = neighbour_list[0]
    receivers = neighbour_list[1]
    sc_gather, sc_scatter = _sc_kernels()

    s_rows, r_rows = sc_gather(node_emb, senders, receivers)

    edge_out, ws, wr = _tc_edge(edge_emb, s_rows, r_rows, e_W1, e_b1, e_W2,
                                e_b2, e_g, e_beta, r_W, r_b, s_W, s_b)

    agg = sc_scatter(ws, wr, senders, receivers)
    sent_pad = agg[:N_PAD]
    recv_pad = agg[N_PAD:]

    node_pad = jnp.pad(node_emb, ((0, N_PAD - N), (0, 0)))
    node_out = _tc_node(node_pad, sent_pad, recv_pad, n_W1, n_b1, n_W2, n_b2,
                        n_g, n_beta)[:N]
    return (node_out, edge_out)


# speedup vs baseline: 3.6209x; 1.5525x over previous
import functools

import jax
import jax.numpy as jnp
from jax import lax
from jax.experimental import pallas as pl
from jax.experimental.pallas import tpu as pltpu
from jax.experimental.pallas import tpu_sc as plsc

N = 10000
E = 320000
C = 128
H = 128

N_PAD = 10240
NW = 32
KG = 40
KS = 80
ROWS_PT = N_PAD // 16
ZROWS = ROWS_PT // 8
NHALF = 2
EPART = E // NHALF


def _pipe(nch, start_fill, wait_fill, start_drain, wait_drain):
    start_fill(0, 0)
    start_fill(1, 1)
    iters = (nch - 2) // 2

    def body(k, carry):
        ci = 2 * k
        wait_fill(0)
        start_drain(ci, 0)
        wait_fill(1)
        start_drain(ci + 1, 1)
        wait_drain(0)
        start_fill(ci + 2, 0)
        wait_drain(1)
        start_fill(ci + 3, 1)
        return carry

    lax.fori_loop(0, iters, body, 0)
    r = 2 * iters
    if nch % 2 == 0:
        wait_fill(0)
        start_drain(r, 0)
        wait_fill(1)
        start_drain(r + 1, 1)
    else:
        wait_fill(0)
        start_drain(r, 0)
        wait_drain(0)
        start_fill(nch - 1, 0)
        wait_fill(1)
        start_drain(r + 1, 1)
        wait_fill(0)
        start_drain(nch - 1, 0)
    wait_drain(0)
    wait_drain(1)


@functools.cache
def _sc_kernels(base, e_part):
    mesh = plsc.VectorSubcoreMesh(core_axis_name="c", subcore_axis_name="s")
    epw = e_part // NW
    nchg = epw // KG
    ept = e_part // 16
    nchs = ept // KS

    @functools.partial(
        pl.kernel,
        out_type=[
            jax.ShapeDtypeStruct((e_part, C), jnp.float32),
            jax.ShapeDtypeStruct((e_part, C), jnp.float32),
        ],
        mesh=mesh,
        scratch_types=[
            pltpu.VMEM((epw,), jnp.int32),
            pltpu.VMEM((epw,), jnp.int32),
            pltpu.VMEM((KG, C), jnp.float32),
            pltpu.VMEM((KG, C), jnp.float32),
            pltpu.VMEM((KG, C), jnp.float32),
            pltpu.VMEM((KG, C), jnp.float32),
            pltpu.SemaphoreType.DMA,
            pltpu.SemaphoreType.DMA,
            pltpu.SemaphoreType.DMA,
            pltpu.SemaphoreType.DMA,
            pltpu.SemaphoreType.DMA,
            pltpu.SemaphoreType.DMA,
            pltpu.SemaphoreType.DMA,
            pltpu.SemaphoreType.DMA,
        ],
    )
    def sc_gather(node_hbm, snd_hbm, rcv_hbm, outs_hbm, outr_hbm,
                  sidx, ridx, sb0, sb1, rb0, rb1,
                  gs0, gs1, gr0, gr1, ws0, ws1, wr0, wr1):
        wid = lax.axis_index("s") * 2 + lax.axis_index("c")
        lbase = wid * epw
        c0 = pltpu.async_copy(snd_hbm.at[pl.ds(base + lbase, epw)], sidx, gs0)
        c1 = pltpu.async_copy(rcv_hbm.at[pl.ds(base + lbase, epw)], ridx, gs1)
        c0.wait()
        c1.wait()

        sbufs = (sb0, sb1)
        rbufs = (rb0, rb1)
        gssem = (gs0, gs1)
        grsem = (gr0, gr1)
        wssem = (ws0, ws1)
        wrsem = (wr0, wr1)

        def start_fill(ci, b):
            pltpu.async_copy(node_hbm.at[sidx.at[pl.ds(ci * KG, KG)]],
                             sbufs[b], gssem[b])
            pltpu.async_copy(node_hbm.at[ridx.at[pl.ds(ci * KG, KG)]],
                             rbufs[b], grsem[b])

        def wait_fill(b):
            pltpu.make_async_copy(node_hbm.at[pl.ds(0, KG)], sbufs[b],
                                  gssem[b]).wait()
            pltpu.make_async_copy(node_hbm.at[pl.ds(0, KG)], rbufs[b],
                                  grsem[b]).wait()

        def start_drain(ci, b):
            off = lbase + ci * KG
            pltpu.async_copy(sbufs[b], outs_hbm.at[pl.ds(off, KG)], wssem[b])
            pltpu.async_copy(rbufs[b], outr_hbm.at[pl.ds(off, KG)], wrsem[b])

        def wait_drain(b):
            pltpu.make_async_copy(sbufs[b], outs_hbm.at[pl.ds(0, KG)],
                                  wssem[b]).wait()
            pltpu.make_async_copy(rbufs[b], outr_hbm.at[pl.ds(0, KG)],
                                  wrsem[b]).wait()

        _pipe(nchg, start_fill, wait_fill, start_drain, wait_drain)

    @functools.partial(
        pl.kernel,
        out_type=jax.ShapeDtypeStruct((2 * N_PAD, C), jnp.float32),
        mesh=mesh,
        scratch_types=[
            pltpu.VMEM_SHARED((N_PAD, C), jnp.float32),
            pltpu.VMEM((KS,), jnp.int32),
            pltpu.VMEM((KS,), jnp.int32),
            pltpu.VMEM((KS, C), jnp.float32),
            pltpu.VMEM((KS, C), jnp.float32),
            pltpu.VMEM((ZROWS, C), jnp.float32),
            pltpu.SemaphoreType.DMA,
            pltpu.SemaphoreType.DMA,
            pltpu.SemaphoreType.DMA,
            pltpu.SemaphoreType.DMA,
            pltpu.SemaphoreType.DMA,
            pltpu.SemaphoreType.DMA,
        ],
    )
    def sc_scatter(ws_hbm, wr_hbm, snd_hbm, rcv_hbm, out_hbm,
                   acc, ib0, ib1, db0, db1, zb,
                   si0, si1, sd0, sd1, sa0, sa1):
        cid = lax.axis_index("c")
        sid = lax.axis_index("s")

        def zrow(i, carry):
            for j in range(C // 16):
                zb[i, pl.ds(j * 16, 16)] = jnp.zeros((16,), jnp.float32)
            return carry

        lax.fori_loop(0, ZROWS, zrow, 0)
        for i in range(ROWS_PT // ZROWS):
            pltpu.sync_copy(
                zb, acc.at[pl.ds(sid * ROWS_PT + i * ZROWS, ZROWS)])
        plsc.subcore_barrier()

        ibufs = (ib0, ib1)
        dbufs = (db0, db1)
        isem = (si0, si1)
        dsem = (sd0, sd1)
        asem = (sa0, sa1)

        def process(data_hbm, idx_hbm):
            def start_fill(ci, b):
                pltpu.async_copy(
                    idx_hbm.at[pl.ds(base + sid * ept + ci * KS, KS)],
                    ibufs[b], isem[b])
                pltpu.async_copy(
                    data_hbm.at[pl.ds(sid * ept + ci * KS, KS)],
                    dbufs[b], dsem[b])

            def wait_fill(b):
                pltpu.make_async_copy(idx_hbm.at[pl.ds(0, KS)], ibufs[b],
                                      isem[b]).wait()
                pltpu.make_async_copy(data_hbm.at[pl.ds(0, KS)], dbufs[b],
                                      dsem[b]).wait()

            def start_drain(ci, b):
                pltpu.async_copy(dbufs[b], acc.at[ibufs[b]], asem[b],
                                 add=True)

            def wait_drain(b):
                pltpu.make_async_copy(dbufs[b], acc.at[pl.ds(0, KS)],
                                      asem[b]).wait()

            _pipe(nchs, start_fill, wait_fill, start_drain, wait_drain)

        @pl.when(cid == 0)
        def _():
            process(ws_hbm, snd_hbm)

        @pl.when(cid == 1)
        def _():
            process(wr_hbm, rcv_hbm)

        plsc.subcore_barrier()
        pltpu.sync_copy(
            acc.at[pl.ds(sid * ROWS_PT, ROWS_PT)],
            out_hbm.at[pl.ds(cid * N_PAD + sid * ROWS_PT, ROWS_PT)])

    return sc_gather, sc_scatter


def _edge_body(edge_ref, s_ref, r_ref, w1a_ref, w1b_ref, w1c_ref, b1_ref,
               w2_ref, b2_ref, g_ref, beta_ref, rw_ref, rb_ref, sw_ref,
               sb_ref, eout_ref, ws_ref, wr_ref):
    edge = edge_ref[...]
    h = (jnp.dot(edge, w1a_ref[...], preferred_element_type=jnp.float32)
         + jnp.dot(s_ref[...], w1b_ref[...], preferred_element_type=jnp.float32)
         + jnp.dot(r_ref[...], w1c_ref[...], preferred_element_type=jnp.float32)
         + b1_ref[...])
    h = h * jax.nn.sigmoid(h)
    m = jnp.dot(h, w2_ref[...], preferred_element_type=jnp.float32) + b2_ref[...]
    mu = jnp.mean(m, axis=-1, keepdims=True)
    var = jnp.mean((m - mu) * (m - mu), axis=-1, keepdims=True)
    nef = (m - mu) * lax.rsqrt(var + 1e-5) * g_ref[...] + beta_ref[...]
    ra = jax.nn.sigmoid(
        jnp.sum(edge * rw_ref[...], axis=-1, keepdims=True) + rb_ref[0, 0])
    sa = jax.nn.sigmoid(
        jnp.sum(edge * sw_ref[...], axis=-1, keepdims=True) + sb_ref[0, 0])
    eout_ref[...] = edge + nef
    ws_ref[...] = nef * sa
    wr_ref[...] = nef * ra


def _tc_edge(block_off, edge_emb, s_rows, r_rows, e_W1, e_b1, e_W2, e_b2,
             e_g, e_beta, r_W, r_b, s_W, s_b):
    B = 640
    grid = (EPART // B,)
    full_row = lambda i: (i + block_off, 0)
    row = lambda i: (i, 0)
    rep = lambda i: (0, 0)
    blk = pl.BlockSpec((B, C), row)
    wspec = pl.BlockSpec((C, H), rep)
    vspec = pl.BlockSpec((1, C), rep)
    sspec = pl.BlockSpec((1, 1), rep)
    return pl.pallas_call(
        _edge_body,
        grid=grid,
        in_specs=[pl.BlockSpec((B, C), full_row), blk, blk,
                  wspec, wspec, wspec, vspec,
                  pl.BlockSpec((H, C), rep), vspec, vspec, vspec,
                  vspec, sspec, vspec, sspec],
        out_specs=[blk, blk, blk],
        out_shape=[jax.ShapeDtypeStruct((EPART, C), jnp.float32)] * 3,
    )(edge_emb, s_rows, r_rows,
      e_W1[0:C], e_W1[C:2 * C], e_W1[2 * C:3 * C], e_b1.reshape(1, H),
      e_W2, e_b2.reshape(1, C), e_g.reshape(1, C), e_beta.reshape(1, C),
      r_W.reshape(1, C), r_b.reshape(1, 1),
      s_W.reshape(1, C), s_b.reshape(1, 1))


def _node_body(node_ref, s0_ref, s1_ref, r0_ref, r1_ref, w1a_ref, w1b_ref,
               w1c_ref, b1_ref, w2_ref, b2_ref, g_ref, beta_ref, out_ref):
    node = node_ref[...]
    snt = s0_ref[...] + s1_ref[...]
    rcv = r0_ref[...] + r1_ref[...]
    h = (jnp.dot(node, w1a_ref[...], preferred_element_type=jnp.float32)
         + jnp.dot(snt, w1b_ref[...], preferred_element_type=jnp.float32)
         + jnp.dot(rcv, w1c_ref[...], preferred_element_type=jnp.float32)
         + b1_ref[...])
    h = h * jax.nn.sigmoid(h)
    m = jnp.dot(h, w2_ref[...], preferred_element_type=jnp.float32) + b2_ref[...]
    mu = jnp.mean(m, axis=-1, keepdims=True)
    var = jnp.mean((m - mu) * (m - mu), axis=-1, keepdims=True)
    nnf = (m - mu) * lax.rsqrt(var + 1e-5) * g_ref[...] + beta_ref[...]
    out_ref[...] = node + nnf


def _tc_node(node_pad, agg0, agg1, n_W1, n_b1, n_W2, n_b2, n_g, n_beta):
    B = 512
    grid = (N_PAD // B,)
    row = lambda i: (i, 0)
    recv_row = lambda i: (i + N_PAD // B, 0)
    rep = lambda i: (0, 0)
    blk = pl.BlockSpec((B, C), row)
    rblk = pl.BlockSpec((B, C), recv_row)
    wspec = pl.BlockSpec((C, H), rep)
    vspec = pl.BlockSpec((1, C), rep)
    return pl.pallas_call(
        _node_body,
        grid=grid,
        in_specs=[blk, blk, blk, rblk, rblk,
                  wspec, wspec, wspec, vspec,
                  pl.BlockSpec((H, C), rep), vspec, vspec, vspec],
        out_specs=blk,
        out_shape=jax.ShapeDtypeStruct((N_PAD, C), jnp.float32),
    )(node_pad, agg0, agg1, agg0, agg1,
      n_W1[0:C], n_W1[C:2 * C], n_W1[2 * C:3 * C], n_b1.reshape(1, H),
      n_W2, n_b2.reshape(1, C), n_g.reshape(1, C), n_beta.reshape(1, C))


def kernel(node_emb, edge_emb, neighbour_list, e_W1, e_b1, e_W2, e_b2, e_g,
           e_beta, n_W1, n_b1, n_W2, n_b2, n_g, n_beta, r_W, r_b, s_W, s_b):
    senders = neighbour_list[0]
    receivers = neighbour_list[1]

    eo = []
    wsr = []
    aggs = []
    for half in range(NHALF):
        gather, _ = _sc_kernels(half * EPART, EPART)
        s_rows, r_rows = gather(node_emb, senders, receivers)
        eo_h, ws_h, wr_h = _tc_edge(half * (EPART // 640), edge_emb, s_rows,
                                    r_rows, e_W1, e_b1, e_W2, e_b2, e_g,
                                    e_beta, r_W, r_b, s_W, s_b)
        eo.append(eo_h)
        wsr.append((ws_h, wr_h))
    for half in range(NHALF):
        _, scatter = _sc_kernels(half * EPART, EPART)
        aggs.append(scatter(wsr[half][0], wsr[half][1], senders, receivers))

    edge_out = jnp.concatenate(eo, axis=0)
    node_pad = jnp.pad(node_emb, ((0, N_PAD - N), (0, 0)))
    node_out = _tc_node(node_pad, aggs[0], aggs[1], n_W1, n_b1, n_W2, n_b2,
                        n_g, n_beta)[:N]
    return (node_out, edge_out)


# speedup vs baseline: 3.7388x; 1.0326x over previous
import functools

import jax
import jax.numpy as jnp
from jax import lax
from jax.experimental import pallas as pl
from jax.experimental.pallas import tpu as pltpu
from jax.experimental.pallas import tpu_sc as plsc

N = 10000
E = 320000
C = 128
H = 128

N_PAD = 10240
NW = 32
KG = 40
KS = 80
ROWS_PT = N_PAD // 16
ZROWS = ROWS_PT // 8
NHALF = 2
EPART = E // NHALF


def _pipe(nch, nbuf, start_fill, wait_fill, start_drain, wait_drain):
    assert nch >= 2 * nbuf
    for b in range(nbuf):
        start_fill(b, b)
    ngroups = nch // nbuf
    rem = nch % nbuf

    def body(k, carry):
        c = nbuf * k
        for b in range(nbuf):
            wait_fill(b)
            start_drain(c + b, b)
            if b >= 1:
                wait_drain(b - 1)
                start_fill(c + nbuf + b - 1, b - 1)
        wait_drain(nbuf - 1)
        start_fill(c + 2 * nbuf - 1, nbuf - 1)
        return carry

    lax.fori_loop(0, ngroups - 1, body, 0)
    c = nbuf * (ngroups - 1)
    for b in range(nbuf):
        wait_fill(b)
        start_drain(c + b, b)
    for j in range(rem):
        ci = nbuf * ngroups + j
        wait_drain(j)
        start_fill(ci, j)
        wait_fill(j)
        start_drain(ci, j)
    for b in range(nbuf):
        wait_drain(b)


@functools.cache
def _sc_kernels(base, e_part):
    mesh = plsc.VectorSubcoreMesh(core_axis_name="c", subcore_axis_name="s")
    epw = e_part // NW
    nchg = epw // KG
    ept = e_part // 16
    nchs = ept // KS

    nbg = 4
    @functools.partial(
        pl.kernel,
        out_type=[
            jax.ShapeDtypeStruct((e_part, C), jnp.float32),
            jax.ShapeDtypeStruct((e_part, C), jnp.float32),
        ],
        mesh=mesh,
        scratch_types=(
            [pltpu.VMEM((epw,), jnp.int32)] * 2
            + [pltpu.VMEM((KG, C), jnp.float32)] * (2 * nbg)
            + [pltpu.SemaphoreType.DMA] * (4 * nbg)
        ),
    )
    def sc_gather(node_hbm, snd_hbm, rcv_hbm, outs_hbm, outr_hbm, *scr):
        sidx, ridx = scr[0], scr[1]
        sbufs = scr[2:2 + nbg]
        rbufs = scr[2 + nbg:2 + 2 * nbg]
        sems = scr[2 + 2 * nbg:]
        gssem = sems[0:nbg]
        grsem = sems[nbg:2 * nbg]
        wssem = sems[2 * nbg:3 * nbg]
        wrsem = sems[3 * nbg:4 * nbg]

        wid = lax.axis_index("s") * 2 + lax.axis_index("c")
        lbase = wid * epw
        c0 = pltpu.async_copy(snd_hbm.at[pl.ds(base + lbase, epw)], sidx,
                              gssem[0])
        c1 = pltpu.async_copy(rcv_hbm.at[pl.ds(base + lbase, epw)], ridx,
                              grsem[0])
        c0.wait()
        c1.wait()

        def start_fill(ci, b):
            pltpu.async_copy(node_hbm.at[sidx.at[pl.ds(ci * KG, KG)]],
                             sbufs[b], gssem[b])
            pltpu.async_copy(node_hbm.at[ridx.at[pl.ds(ci * KG, KG)]],
                             rbufs[b], grsem[b])

        def wait_fill(b):
            pltpu.make_async_copy(node_hbm.at[pl.ds(0, KG)], sbufs[b],
                                  gssem[b]).wait()
            pltpu.make_async_copy(node_hbm.at[pl.ds(0, KG)], rbufs[b],
                                  grsem[b]).wait()

        def start_drain(ci, b):
            off = lbase + ci * KG
            pltpu.async_copy(sbufs[b], outs_hbm.at[pl.ds(off, KG)], wssem[b])
            pltpu.async_copy(rbufs[b], outr_hbm.at[pl.ds(off, KG)], wrsem[b])

        def wait_drain(b):
            pltpu.make_async_copy(sbufs[b], outs_hbm.at[pl.ds(0, KG)],
                                  wssem[b]).wait()
            pltpu.make_async_copy(rbufs[b], outr_hbm.at[pl.ds(0, KG)],
                                  wrsem[b]).wait()

        _pipe(nchg, nbg, start_fill, wait_fill, start_drain, wait_drain)

    nbs = 3
    @functools.partial(
        pl.kernel,
        out_type=jax.ShapeDtypeStruct((2 * N_PAD, C), jnp.float32),
        mesh=mesh,
        scratch_types=(
            [pltpu.VMEM_SHARED((N_PAD, C), jnp.float32)]
            + [pltpu.VMEM((KS,), jnp.int32)] * nbs
            + [pltpu.VMEM((KS, C), jnp.float32)] * nbs
            + [pltpu.VMEM((ZROWS, C), jnp.float32)]
            + [pltpu.SemaphoreType.DMA] * (3 * nbs)
        ),
    )
    def sc_scatter(ws_hbm, wr_hbm, snd_hbm, rcv_hbm, out_hbm, *scr):
        acc = scr[0]
        ibufs = scr[1:1 + nbs]
        dbufs = scr[1 + nbs:1 + 2 * nbs]
        zb = scr[1 + 2 * nbs]
        sems = scr[2 + 2 * nbs:]
        isem = sems[0:nbs]
        dsem = sems[nbs:2 * nbs]
        asem = sems[2 * nbs:3 * nbs]

        cid = lax.axis_index("c")
        sid = lax.axis_index("s")

        def zrow(i, carry):
            for j in range(C // 16):
                zb[i, pl.ds(j * 16, 16)] = jnp.zeros((16,), jnp.float32)
            return carry

        lax.fori_loop(0, ZROWS, zrow, 0)
        for i in range(ROWS_PT // ZROWS):
            pltpu.sync_copy(
                zb, acc.at[pl.ds(sid * ROWS_PT + i * ZROWS, ZROWS)])
        plsc.subcore_barrier()

        def process(data_hbm, idx_hbm):
            def start_fill(ci, b):
                pltpu.async_copy(
                    idx_hbm.at[pl.ds(base + sid * ept + ci * KS, KS)],
                    ibufs[b], isem[b])
                pltpu.async_copy(
                    data_hbm.at[pl.ds(sid * ept + ci * KS, KS)],
                    dbufs[b], dsem[b])

            def wait_fill(b):
                pltpu.make_async_copy(idx_hbm.at[pl.ds(0, KS)], ibufs[b],
                                      isem[b]).wait()
                pltpu.make_async_copy(data_hbm.at[pl.ds(0, KS)], dbufs[b],
                                      dsem[b]).wait()

            def start_drain(ci, b):
                pltpu.async_copy(dbufs[b], acc.at[ibufs[b]], asem[b],
                                 add=True)

            def wait_drain(b):
                pltpu.make_async_copy(dbufs[b], acc.at[pl.ds(0, KS)],
                                      asem[b]).wait()

            _pipe(nchs, nbs, start_fill, wait_fill, start_drain, wait_drain)

        @pl.when(cid == 0)
        def _():
            process(ws_hbm, snd_hbm)

        @pl.when(cid == 1)
        def _():
            process(wr_hbm, rcv_hbm)

        plsc.subcore_barrier()
        pltpu.sync_copy(
            acc.at[pl.ds(sid * ROWS_PT, ROWS_PT)],
            out_hbm.at[pl.ds(cid * N_PAD + sid * ROWS_PT, ROWS_PT)])

    return sc_gather, sc_scatter


def _edge_body(edge_ref, s_ref, r_ref, w1a_ref, w1b_ref, w1c_ref, b1_ref,
               w2_ref, b2_ref, g_ref, beta_ref, rw_ref, rb_ref, sw_ref,
               sb_ref, eout_ref, ws_ref, wr_ref):
    edge = edge_ref[...]
    h = (jnp.dot(edge, w1a_ref[...], preferred_element_type=jnp.float32)
         + jnp.dot(s_ref[...], w1b_ref[...], preferred_element_type=jnp.float32)
         + jnp.dot(r_ref[...], w1c_ref[...], preferred_element_type=jnp.float32)
         + b1_ref[...])
    h = h * jax.nn.sigmoid(h)
    m = jnp.dot(h, w2_ref[...], preferred_element_type=jnp.float32) + b2_ref[...]
    mu = jnp.mean(m, axis=-1, keepdims=True)
    var = jnp.mean((m - mu) * (m - mu), axis=-1, keepdims=True)
    nef = (m - mu) * lax.rsqrt(var + 1e-5) * g_ref[...] + beta_ref[...]
    ra = jax.nn.sigmoid(
        jnp.sum(edge * rw_ref[...], axis=-1, keepdims=True) + rb_ref[0, 0])
    sa = jax.nn.sigmoid(
        jnp.sum(edge * sw_ref[...], axis=-1, keepdims=True) + sb_ref[0, 0])
    eout_ref[...] = edge + nef
    ws_ref[...] = nef * sa
    wr_ref[...] = nef * ra


def _tc_edge(block_off, edge_emb, s_rows, r_rows, e_W1, e_b1, e_W2, e_b2,
             e_g, e_beta, r_W, r_b, s_W, s_b):
    B = 640
    grid = (EPART // B,)
    full_row = lambda i: (i + block_off, 0)
    row = lambda i: (i, 0)
    rep = lambda i: (0, 0)
    blk = pl.BlockSpec((B, C), row)
    wspec = pl.BlockSpec((C, H), rep)
    vspec = pl.BlockSpec((1, C), rep)
    sspec = pl.BlockSpec((1, 1), rep)
    return pl.pallas_call(
        _edge_body,
        grid=grid,
        in_specs=[pl.BlockSpec((B, C), full_row), blk, blk,
                  wspec, wspec, wspec, vspec,
                  pl.BlockSpec((H, C), rep), vspec, vspec, vspec,
                  vspec, sspec, vspec, sspec],
        out_specs=[blk, blk, blk],
        out_shape=[jax.ShapeDtypeStruct((EPART, C), jnp.float32)] * 3,
    )(edge_emb, s_rows, r_rows,
      e_W1[0:C], e_W1[C:2 * C], e_W1[2 * C:3 * C], e_b1.reshape(1, H),
      e_W2, e_b2.reshape(1, C), e_g.reshape(1, C), e_beta.reshape(1, C),
      r_W.reshape(1, C), r_b.reshape(1, 1),
      s_W.reshape(1, C), s_b.reshape(1, 1))


def _node_body(node_ref, s0_ref, s1_ref, r0_ref, r1_ref, w1a_ref, w1b_ref,
               w1c_ref, b1_ref, w2_ref, b2_ref, g_ref, beta_ref, out_ref):
    node = node_ref[...]
    snt = s0_ref[...] + s1_ref[...]
    rcv = r0_ref[...] + r1_ref[...]
    h = (jnp.dot(node, w1a_ref[...], preferred_element_type=jnp.float32)
         + jnp.dot(snt, w1b_ref[...], preferred_element_type=jnp.float32)
         + jnp.dot(rcv, w1c_ref[...], preferred_element_type=jnp.float32)
         + b1_ref[...])
    h = h * jax.nn.sigmoid(h)
    m = jnp.dot(h, w2_ref[...], preferred_element_type=jnp.float32) + b2_ref[...]
    mu = jnp.mean(m, axis=-1, keepdims=True)
    var = jnp.mean((m - mu) * (m - mu), axis=-1, keepdims=True)
    nnf = (m - mu) * lax.rsqrt(var + 1e-5) * g_ref[...] + beta_ref[...]
    out_ref[...] = node + nnf


def _tc_node(node_pad, agg0, agg1, n_W1, n_b1, n_W2, n_b2, n_g, n_beta):
    B = 512
    grid = (N_PAD // B,)
    row = lambda i: (i, 0)
    recv_row = lambda i: (i + N_PAD // B, 0)
    rep = lambda i: (0, 0)
    blk = pl.BlockSpec((B, C), row)
    rblk = pl.BlockSpec((B, C), recv_row)
    wspec = pl.BlockSpec((C, H), rep)
    vspec = pl.BlockSpec((1, C), rep)
    return pl.pallas_call(
        _node_body,
        grid=grid,
        in_specs=[blk, blk, blk, rblk, rblk,
                  wspec, wspec, wspec, vspec,
                  pl.BlockSpec((H, C), rep), vspec, vspec, vspec],
        out_specs=blk,
        out_shape=jax.ShapeDtypeStruct((N_PAD, C), jnp.float32),
    )(node_pad, agg0, agg1, agg0, agg1,
      n_W1[0:C], n_W1[C:2 * C], n_W1[2 * C:3 * C], n_b1.reshape(1, H),
      n_W2, n_b2.reshape(1, C), n_g.reshape(1, C), n_beta.reshape(1, C))


def kernel(node_emb, edge_emb, neighbour_list, e_W1, e_b1, e_W2, e_b2, e_g,
           e_beta, n_W1, n_b1, n_W2, n_b2, n_g, n_beta, r_W, r_b, s_W, s_b):
    senders = neighbour_list[0]
    receivers = neighbour_list[1]

    eo = []
    wsr = []
    aggs = []
    for half in range(NHALF):
        gather, _ = _sc_kernels(half * EPART, EPART)
        s_rows, r_rows = gather(node_emb, senders, receivers)
        eo_h, ws_h, wr_h = _tc_edge(half * (EPART // 640), edge_emb, s_rows,
                                    r_rows, e_W1, e_b1, e_W2, e_b2, e_g,
                                    e_beta, r_W, r_b, s_W, s_b)
        eo.append(eo_h)
        wsr.append((ws_h, wr_h))
    for half in range(NHALF):
        _, scatter = _sc_kernels(half * EPART, EPART)
        aggs.append(scatter(wsr[half][0], wsr[half][1], senders, receivers))

    edge_out = jnp.concatenate(eo, axis=0)
    node_pad = jnp.pad(node_emb, ((0, N_PAD - N), (0, 0)))
    node_out = _tc_node(node_pad, aggs[0], aggs[1], n_W1, n_b1, n_W2, n_b2,
                        n_g, n_beta)[:N]
    return (node_out, edge_out)


# speedup vs baseline: 4.5467x; 1.2161x over previous
import functools

import jax
import jax.numpy as jnp
from jax import lax
from jax.experimental import pallas as pl
from jax.experimental.pallas import tpu as pltpu
from jax.experimental.pallas import tpu_sc as plsc

N = 10000
E = 320000
C = 128
H = 128

N_PAD = 10240
NW = 32
KG = 40
KS = 80
ROWS_PT = N_PAD // 16
ZROWS = ROWS_PT // 8
NHALF = 2
EPART = E // NHALF


def _pipe(nch, nbuf, start_fill, wait_fill, start_drain, wait_drain):
    assert nch >= 2 * nbuf
    for b in range(nbuf):
        start_fill(b, b)
    ngroups = nch // nbuf
    rem = nch % nbuf

    def body(k, carry):
        c = nbuf * k
        for b in range(nbuf):
            wait_fill(b)
            start_drain(c + b, b)
            if b >= 1:
                wait_drain(b - 1)
                start_fill(c + nbuf + b - 1, b - 1)
        wait_drain(nbuf - 1)
        start_fill(c + 2 * nbuf - 1, nbuf - 1)
        return carry

    lax.fori_loop(0, ngroups - 1, body, 0)
    c = nbuf * (ngroups - 1)
    for b in range(nbuf):
        wait_fill(b)
        start_drain(c + b, b)
    for j in range(rem):
        ci = nbuf * ngroups + j
        wait_drain(j)
        start_fill(ci, j)
        wait_fill(j)
        start_drain(ci, j)
    for b in range(nbuf):
        wait_drain(b)


@functools.cache
def _sc_kernels(base, e_part):
    mesh = plsc.VectorSubcoreMesh(core_axis_name="c", subcore_axis_name="s")
    epw = e_part // NW
    nchg = epw // KG
    ept = e_part // 16
    nchs = ept // KS

    nbg = 4
    @functools.partial(
        pl.kernel,
        out_type=[
            jax.ShapeDtypeStruct((e_part, C), jnp.float32),
            jax.ShapeDtypeStruct((e_part, C), jnp.float32),
        ],
        mesh=mesh,
        scratch_types=(
            [pltpu.VMEM((epw,), jnp.int32)] * 2
            + [pltpu.VMEM((KG, C), jnp.float32)] * (2 * nbg)
            + [pltpu.SemaphoreType.DMA] * (4 * nbg)
        ),
    )
    def sc_gather(node_hbm, snd_hbm, rcv_hbm, outs_hbm, outr_hbm, *scr):
        sidx, ridx = scr[0], scr[1]
        sbufs = scr[2:2 + nbg]
        rbufs = scr[2 + nbg:2 + 2 * nbg]
        sems = scr[2 + 2 * nbg:]
        gssem = sems[0:nbg]
        grsem = sems[nbg:2 * nbg]
        wssem = sems[2 * nbg:3 * nbg]
        wrsem = sems[3 * nbg:4 * nbg]

        wid = lax.axis_index("s") * 2 + lax.axis_index("c")
        lbase = wid * epw
        c0 = pltpu.async_copy(snd_hbm.at[pl.ds(base + lbase, epw)], sidx,
                              gssem[0])
        c1 = pltpu.async_copy(rcv_hbm.at[pl.ds(base + lbase, epw)], ridx,
                              grsem[0])
        c0.wait()
        c1.wait()

        def start_fill(ci, b):
            pltpu.async_copy(node_hbm.at[sidx.at[pl.ds(ci * KG, KG)]],
                             sbufs[b], gssem[b])
            pltpu.async_copy(node_hbm.at[ridx.at[pl.ds(ci * KG, KG)]],
                             rbufs[b], grsem[b])

        def wait_fill(b):
            pltpu.make_async_copy(node_hbm.at[pl.ds(0, KG)], sbufs[b],
                                  gssem[b]).wait()
            pltpu.make_async_copy(node_hbm.at[pl.ds(0, KG)], rbufs[b],
                                  grsem[b]).wait()

        def start_drain(ci, b):
            off = lbase + ci * KG
            pltpu.async_copy(sbufs[b], outs_hbm.at[pl.ds(off, KG)], wssem[b])
            pltpu.async_copy(rbufs[b], outr_hbm.at[pl.ds(off, KG)], wrsem[b])

        def wait_drain(b):
            pltpu.make_async_copy(sbufs[b], outs_hbm.at[pl.ds(0, KG)],
                                  wssem[b]).wait()
            pltpu.make_async_copy(rbufs[b], outr_hbm.at[pl.ds(0, KG)],
                                  wrsem[b]).wait()

        _pipe(nchg, nbg, start_fill, wait_fill, start_drain, wait_drain)

    nbs = 3
    @functools.partial(
        pl.kernel,
        out_type=jax.ShapeDtypeStruct((2 * N_PAD, C), jnp.float32),
        mesh=mesh,
        scratch_types=(
            [pltpu.VMEM_SHARED((N_PAD, C), jnp.float32)]
            + [pltpu.VMEM((KS,), jnp.int32)] * nbs
            + [pltpu.VMEM((KS, C), jnp.float32)] * nbs
            + [pltpu.VMEM((ZROWS, C), jnp.float32)]
            + [pltpu.SemaphoreType.DMA] * (3 * nbs)
        ),
    )
    def sc_scatter(ws_hbm, wr_hbm, snd_hbm, rcv_hbm, out_hbm, *scr):
        acc = scr[0]
        ibufs = scr[1:1 + nbs]
        dbufs = scr[1 + nbs:1 + 2 * nbs]
        zb = scr[1 + 2 * nbs]
        sems = scr[2 + 2 * nbs:]
        isem = sems[0:nbs]
        dsem = sems[nbs:2 * nbs]
        asem = sems[2 * nbs:3 * nbs]

        cid = lax.axis_index("c")
        sid = lax.axis_index("s")

        def zrow(i, carry):
            for j in range(C // 16):
                zb[i, pl.ds(j * 16, 16)] = jnp.zeros((16,), jnp.float32)
            return carry

        lax.fori_loop(0, ZROWS, zrow, 0)
        for i in range(ROWS_PT // ZROWS):
            pltpu.sync_copy(
                zb, acc.at[pl.ds(sid * ROWS_PT + i * ZROWS, ZROWS)])
        plsc.subcore_barrier()

        def process(data_hbm, idx_hbm):
            def start_fill(ci, b):
                pltpu.async_copy(
                    idx_hbm.at[pl.ds(base + sid * ept + ci * KS, KS)],
                    ibufs[b], isem[b])
                pltpu.async_copy(
                    data_hbm.at[pl.ds(sid * ept + ci * KS, KS)],
                    dbufs[b], dsem[b])

            def wait_fill(b):
                pltpu.make_async_copy(idx_hbm.at[pl.ds(0, KS)], ibufs[b],
                                      isem[b]).wait()
                pltpu.make_async_copy(data_hbm.at[pl.ds(0, KS)], dbufs[b],
                                      dsem[b]).wait()

            def start_drain(ci, b):
                pltpu.async_copy(dbufs[b], acc.at[ibufs[b]], asem[b],
                                 add=True)

            def wait_drain(b):
                pltpu.make_async_copy(dbufs[b], acc.at[pl.ds(0, KS)],
                                      asem[b]).wait()

            _pipe(nchs, nbs, start_fill, wait_fill, start_drain, wait_drain)

        @pl.when(cid == 0)
        def _():
            process(ws_hbm, snd_hbm)

        @pl.when(cid == 1)
        def _():
            process(wr_hbm, rcv_hbm)

        plsc.subcore_barrier()
        pltpu.sync_copy(
            acc.at[pl.ds(sid * ROWS_PT, ROWS_PT)],
            out_hbm.at[pl.ds(cid * N_PAD + sid * ROWS_PT, ROWS_PT)])

    return sc_gather, sc_scatter


def _edge_body(edge_ref, s_ref, r_ref, w1a_ref, w1b_ref, w1c_ref, b1_ref,
               w2_ref, b2_ref, g_ref, beta_ref, rw_ref, rb_ref, sw_ref,
               sb_ref, eout_ref, ws_ref, wr_ref):
    edge = edge_ref[...]
    h = (jnp.dot(edge, w1a_ref[...], preferred_element_type=jnp.float32)
         + jnp.dot(s_ref[...], w1b_ref[...], preferred_element_type=jnp.float32)
         + jnp.dot(r_ref[...], w1c_ref[...], preferred_element_type=jnp.float32)
         + b1_ref[...])
    h = h * jax.nn.sigmoid(h)
    m = jnp.dot(h, w2_ref[...], preferred_element_type=jnp.float32) + b2_ref[...]
    mu = jnp.mean(m, axis=-1, keepdims=True)
    var = jnp.mean((m - mu) * (m - mu), axis=-1, keepdims=True)
    nef = (m - mu) * lax.rsqrt(var + 1e-5) * g_ref[...] + beta_ref[...]
    ra = jax.nn.sigmoid(
        jnp.sum(edge * rw_ref[...], axis=-1, keepdims=True) + rb_ref[0, 0])
    sa = jax.nn.sigmoid(
        jnp.sum(edge * sw_ref[...], axis=-1, keepdims=True) + sb_ref[0, 0])
    eout_ref[...] = edge + nef
    ws_ref[...] = nef * sa
    wr_ref[...] = nef * ra


def _tc_edge(block_off, edge_emb, s_rows, r_rows, e_W1, e_b1, e_W2, e_b2,
             e_g, e_beta, r_W, r_b, s_W, s_b):
    B = 2000
    grid = (EPART // B,)
    full_row = lambda i: (i + block_off, 0)
    row = lambda i: (i, 0)
    rep = lambda i: (0, 0)
    blk = pl.BlockSpec((B, C), row)
    wspec = pl.BlockSpec((C, H), rep)
    vspec = pl.BlockSpec((1, C), rep)
    sspec = pl.BlockSpec((1, 1), rep)
    return pl.pallas_call(
        _edge_body,
        grid=grid,
        in_specs=[pl.BlockSpec((B, C), full_row), blk, blk,
                  wspec, wspec, wspec, vspec,
                  pl.BlockSpec((H, C), rep), vspec, vspec, vspec,
                  vspec, sspec, vspec, sspec],
        out_specs=[blk, blk, blk],
        out_shape=[jax.ShapeDtypeStruct((EPART, C), jnp.float32)] * 3,
    )(edge_emb, s_rows, r_rows,
      e_W1[0:C], e_W1[C:2 * C], e_W1[2 * C:3 * C], e_b1.reshape(1, H),
      e_W2, e_b2.reshape(1, C), e_g.reshape(1, C), e_beta.reshape(1, C),
      r_W.reshape(1, C), r_b.reshape(1, 1),
      s_W.reshape(1, C), s_b.reshape(1, 1))


def _node_body(node_ref, s0_ref, s1_ref, r0_ref, r1_ref, w1a_ref, w1b_ref,
               w1c_ref, b1_ref, w2_ref, b2_ref, g_ref, beta_ref, out_ref):
    node = node_ref[...]
    snt = s0_ref[...] + s1_ref[...]
    rcv = r0_ref[...] + r1_ref[...]
    h = (jnp.dot(node, w1a_ref[...], preferred_element_type=jnp.float32)
         + jnp.dot(snt, w1b_ref[...], preferred_element_type=jnp.float32)
         + jnp.dot(rcv, w1c_ref[...], preferred_element_type=jnp.float32)
         + b1_ref[...])
    h = h * jax.nn.sigmoid(h)
    m = jnp.dot(h, w2_ref[...], preferred_element_type=jnp.float32) + b2_ref[...]
    mu = jnp.mean(m, axis=-1, keepdims=True)
    var = jnp.mean((m - mu) * (m - mu), axis=-1, keepdims=True)
    nnf = (m - mu) * lax.rsqrt(var + 1e-5) * g_ref[...] + beta_ref[...]
    out_ref[...] = node + nnf


def _tc_node(node_pad, agg0, agg1, n_W1, n_b1, n_W2, n_b2, n_g, n_beta):
    B = 512
    grid = (N_PAD // B,)
    row = lambda i: (i, 0)
    recv_row = lambda i: (i + N_PAD // B, 0)
    rep = lambda i: (0, 0)
    blk = pl.BlockSpec((B, C), row)
    rblk = pl.BlockSpec((B, C), recv_row)
    wspec = pl.BlockSpec((C, H), rep)
    vspec = pl.BlockSpec((1, C), rep)
    return pl.pallas_call(
        _node_body,
        grid=grid,
        in_specs=[blk, blk, blk, rblk, rblk,
                  wspec, wspec, wspec, vspec,
                  pl.BlockSpec((H, C), rep), vspec, vspec, vspec],
        out_specs=blk,
        out_shape=jax.ShapeDtypeStruct((N_PAD, C), jnp.float32),
    )(node_pad, agg0, agg1, agg0, agg1,
      n_W1[0:C], n_W1[C:2 * C], n_W1[2 * C:3 * C], n_b1.reshape(1, H),
      n_W2, n_b2.reshape(1, C), n_g.reshape(1, C), n_beta.reshape(1, C))


def kernel(node_emb, edge_emb, neighbour_list, e_W1, e_b1, e_W2, e_b2, e_g,
           e_beta, n_W1, n_b1, n_W2, n_b2, n_g, n_beta, r_W, r_b, s_W, s_b):
    senders = neighbour_list[0]
    receivers = neighbour_list[1]

    eo = []
    wsr = []
    aggs = []
    for half in range(NHALF):
        gather, _ = _sc_kernels(half * EPART, EPART)
        s_rows, r_rows = gather(node_emb, senders, receivers)
        eo_h, ws_h, wr_h = _tc_edge(half * (EPART // 2000), edge_emb, s_rows,
                                    r_rows, e_W1, e_b1, e_W2, e_b2, e_g,
                                    e_beta, r_W, r_b, s_W, s_b)
        eo.append(eo_h)
        wsr.append((ws_h, wr_h))
    for half in range(NHALF):
        _, scatter = _sc_kernels(half * EPART, EPART)
        aggs.append(scatter(wsr[half][0], wsr[half][1], senders, receivers))

    edge_out = jnp.concatenate(eo, axis=0)
    node_pad = jnp.pad(node_emb, ((0, N_PAD - N), (0, 0)))
    node_out = _tc_node(node_pad, aggs[0], aggs[1], n_W1, n_b1, n_W2, n_b2,
                        n_g, n_beta)[:N]
    return (node_out, edge_out)


# speedup vs baseline: 4.6745x; 1.0281x over previous
import functools

import jax
import jax.numpy as jnp
from jax import lax
from jax.experimental import pallas as pl
from jax.experimental.pallas import tpu as pltpu
from jax.experimental.pallas import tpu_sc as plsc

N = 10000
E = 320000
C = 128
H = 128

N_PAD = 10240
NW = 32
KG = 40
KS = 80
ROWS_PT = N_PAD // 16
ZROWS = ROWS_PT // 8
NHALF = 2
EPART = E // NHALF


def _pipe(nch, nbuf, start_fill, wait_fill, start_drain, wait_drain):
    assert nch >= 2 * nbuf
    for b in range(nbuf):
        start_fill(b, b)
    ngroups = nch // nbuf
    rem = nch % nbuf

    def body(k, carry):
        c = nbuf * k
        for b in range(nbuf):
            wait_fill(b)
            start_drain(c + b, b)
            if b >= 1:
                wait_drain(b - 1)
                start_fill(c + nbuf + b - 1, b - 1)
        wait_drain(nbuf - 1)
        start_fill(c + 2 * nbuf - 1, nbuf - 1)
        return carry

    lax.fori_loop(0, ngroups - 1, body, 0)
    c = nbuf * (ngroups - 1)
    for b in range(nbuf):
        wait_fill(b)
        start_drain(c + b, b)
    for j in range(rem):
        ci = nbuf * ngroups + j
        wait_drain(j)
        start_fill(ci, j)
        wait_fill(j)
        start_drain(ci, j)
    for b in range(nbuf):
        wait_drain(b)


@functools.cache
def _sc_kernels(base, e_part):
    mesh = plsc.VectorSubcoreMesh(core_axis_name="c", subcore_axis_name="s")
    epw = e_part // NW
    nchg = epw // KG
    ept = e_part // 16
    nchs = ept // KS

    nbg = 4
    @functools.partial(
        pl.kernel,
        out_type=[
            jax.ShapeDtypeStruct((e_part, C), jnp.float32),
            jax.ShapeDtypeStruct((e_part, C), jnp.float32),
        ],
        mesh=mesh,
        scratch_types=(
            [pltpu.VMEM((epw,), jnp.int32)] * 2
            + [pltpu.VMEM((KG, C), jnp.float32)] * (2 * nbg)
            + [pltpu.SemaphoreType.DMA] * (4 * nbg)
        ),
    )
    def sc_gather(node_hbm, snd_hbm, rcv_hbm, outs_hbm, outr_hbm, *scr):
        sidx, ridx = scr[0], scr[1]
        sbufs = scr[2:2 + nbg]
        rbufs = scr[2 + nbg:2 + 2 * nbg]
        sems = scr[2 + 2 * nbg:]
        gssem = sems[0:nbg]
        grsem = sems[nbg:2 * nbg]
        wssem = sems[2 * nbg:3 * nbg]
        wrsem = sems[3 * nbg:4 * nbg]

        wid = lax.axis_index("s") * 2 + lax.axis_index("c")
        lbase = wid * epw
        c0 = pltpu.async_copy(snd_hbm.at[pl.ds(base + lbase, epw)], sidx,
                              gssem[0])
        c1 = pltpu.async_copy(rcv_hbm.at[pl.ds(base + lbase, epw)], ridx,
                              grsem[0])
        c0.wait()
        c1.wait()

        def start_fill(ci, b):
            pltpu.async_copy(node_hbm.at[sidx.at[pl.ds(ci * KG, KG)]],
                             sbufs[b], gssem[b])
            pltpu.async_copy(node_hbm.at[ridx.at[pl.ds(ci * KG, KG)]],
                             rbufs[b], grsem[b])

        def wait_fill(b):
            pltpu.make_async_copy(node_hbm.at[pl.ds(0, KG)], sbufs[b],
                                  gssem[b]).wait()
            pltpu.make_async_copy(node_hbm.at[pl.ds(0, KG)], rbufs[b],
                                  grsem[b]).wait()

        def start_drain(ci, b):
            off = lbase + ci * KG
            pltpu.async_copy(sbufs[b], outs_hbm.at[pl.ds(off, KG)], wssem[b])
            pltpu.async_copy(rbufs[b], outr_hbm.at[pl.ds(off, KG)], wrsem[b])

        def wait_drain(b):
            pltpu.make_async_copy(sbufs[b], outs_hbm.at[pl.ds(0, KG)],
                                  wssem[b]).wait()
            pltpu.make_async_copy(rbufs[b], outr_hbm.at[pl.ds(0, KG)],
                                  wrsem[b]).wait()

        _pipe(nchg, nbg, start_fill, wait_fill, start_drain, wait_drain)

    nbs = 3
    @functools.partial(
        pl.kernel,
        out_type=jax.ShapeDtypeStruct((2 * N_PAD, C), jnp.float32),
        mesh=mesh,
        scratch_types=(
            [pltpu.VMEM_SHARED((N_PAD, C), jnp.float32)]
            + [pltpu.VMEM((KS,), jnp.int32)] * nbs
            + [pltpu.VMEM((KS, C), jnp.float32)] * nbs
            + [pltpu.VMEM((ZROWS, C), jnp.float32)]
            + [pltpu.SemaphoreType.DMA] * (3 * nbs)
        ),
    )
    def sc_scatter(ws_hbm, wr_hbm, snd_hbm, rcv_hbm, out_hbm, *scr):
        acc = scr[0]
        ibufs = scr[1:1 + nbs]
        dbufs = scr[1 + nbs:1 + 2 * nbs]
        zb = scr[1 + 2 * nbs]
        sems = scr[2 + 2 * nbs:]
        isem = sems[0:nbs]
        dsem = sems[nbs:2 * nbs]
        asem = sems[2 * nbs:3 * nbs]

        cid = lax.axis_index("c")
        sid = lax.axis_index("s")

        def zrow(i, carry):
            for j in range(C // 16):
                zb[i, pl.ds(j * 16, 16)] = jnp.zeros((16,), jnp.float32)
            return carry

        lax.fori_loop(0, ZROWS, zrow, 0)
        for i in range(ROWS_PT // ZROWS):
            pltpu.sync_copy(
                zb, acc.at[pl.ds(sid * ROWS_PT + i * ZROWS, ZROWS)])
        plsc.subcore_barrier()

        def process(data_hbm, idx_hbm):
            def start_fill(ci, b):
                pltpu.async_copy(
                    idx_hbm.at[pl.ds(base + sid * ept + ci * KS, KS)],
                    ibufs[b], isem[b])
                pltpu.async_copy(
                    data_hbm.at[pl.ds(sid * ept + ci * KS, KS)],
                    dbufs[b], dsem[b])

            def wait_fill(b):
                pltpu.make_async_copy(idx_hbm.at[pl.ds(0, KS)], ibufs[b],
                                      isem[b]).wait()
                pltpu.make_async_copy(data_hbm.at[pl.ds(0, KS)], dbufs[b],
                                      dsem[b]).wait()

            def start_drain(ci, b):
                pltpu.async_copy(dbufs[b], acc.at[ibufs[b]], asem[b],
                                 add=True)

            def wait_drain(b):
                pltpu.make_async_copy(dbufs[b], acc.at[pl.ds(0, KS)],
                                      asem[b]).wait()

            _pipe(nchs, nbs, start_fill, wait_fill, start_drain, wait_drain)

        @pl.when(cid == 0)
        def _():
            process(ws_hbm, snd_hbm)

        @pl.when(cid == 1)
        def _():
            process(wr_hbm, rcv_hbm)

        plsc.subcore_barrier()
        pltpu.sync_copy(
            acc.at[pl.ds(sid * ROWS_PT, ROWS_PT)],
            out_hbm.at[pl.ds(cid * N_PAD + sid * ROWS_PT, ROWS_PT)])

    return sc_gather, sc_scatter


def _edge_body(edge_ref, s_ref, r_ref, w1a_ref, w1b_ref, w1c_ref, b1_ref,
               w2_ref, b2_ref, g_ref, beta_ref, rw_ref, rb_ref, sw_ref,
               sb_ref, eout_ref, ws_ref, wr_ref):
    edge = edge_ref[...]
    h = (jnp.dot(edge, w1a_ref[...], preferred_element_type=jnp.float32)
         + jnp.dot(s_ref[...], w1b_ref[...], preferred_element_type=jnp.float32)
         + jnp.dot(r_ref[...], w1c_ref[...], preferred_element_type=jnp.float32)
         + b1_ref[...])
    h = h * jax.nn.sigmoid(h)
    m = jnp.dot(h, w2_ref[...], preferred_element_type=jnp.float32) + b2_ref[...]
    mu = jnp.mean(m, axis=-1, keepdims=True)
    var = jnp.mean((m - mu) * (m - mu), axis=-1, keepdims=True)
    nef = (m - mu) * lax.rsqrt(var + 1e-5) * g_ref[...] + beta_ref[...]
    ra = jax.nn.sigmoid(
        jnp.sum(edge * rw_ref[...], axis=-1, keepdims=True) + rb_ref[0, 0])
    sa = jax.nn.sigmoid(
        jnp.sum(edge * sw_ref[...], axis=-1, keepdims=True) + sb_ref[0, 0])
    eout_ref[...] = edge + nef
    ws_ref[...] = nef * sa
    wr_ref[...] = nef * ra


def _tc_edge(block_off, edge_emb, s_rows, r_rows, e_W1, e_b1, e_W2, e_b2,
             e_g, e_beta, r_W, r_b, s_W, s_b):
    B = 4000
    grid = (EPART // B,)
    full_row = lambda i: (i + block_off, 0)
    row = lambda i: (i, 0)
    rep = lambda i: (0, 0)
    blk = pl.BlockSpec((B, C), row)
    wspec = pl.BlockSpec((C, H), rep)
    vspec = pl.BlockSpec((1, C), rep)
    sspec = pl.BlockSpec((1, 1), rep)
    return pl.pallas_call(
        _edge_body,
        grid=grid,
        in_specs=[pl.BlockSpec((B, C), full_row), blk, blk,
                  wspec, wspec, wspec, vspec,
                  pl.BlockSpec((H, C), rep), vspec, vspec, vspec,
                  vspec, sspec, vspec, sspec],
        out_specs=[blk, blk, blk],
        out_shape=[jax.ShapeDtypeStruct((EPART, C), jnp.float32)] * 3,
    )(edge_emb, s_rows, r_rows,
      e_W1[0:C], e_W1[C:2 * C], e_W1[2 * C:3 * C], e_b1.reshape(1, H),
      e_W2, e_b2.reshape(1, C), e_g.reshape(1, C), e_beta.reshape(1, C),
      r_W.reshape(1, C), r_b.reshape(1, 1),
      s_W.reshape(1, C), s_b.reshape(1, 1))


def _node_body(node_ref, s0_ref, s1_ref, r0_ref, r1_ref, w1a_ref, w1b_ref,
               w1c_ref, b1_ref, w2_ref, b2_ref, g_ref, beta_ref, out_ref):
    node = node_ref[...]
    snt = s0_ref[...] + s1_ref[...]
    rcv = r0_ref[...] + r1_ref[...]
    h = (jnp.dot(node, w1a_ref[...], preferred_element_type=jnp.float32)
         + jnp.dot(snt, w1b_ref[...], preferred_element_type=jnp.float32)
         + jnp.dot(rcv, w1c_ref[...], preferred_element_type=jnp.float32)
         + b1_ref[...])
    h = h * jax.nn.sigmoid(h)
    m = jnp.dot(h, w2_ref[...], preferred_element_type=jnp.float32) + b2_ref[...]
    mu = jnp.mean(m, axis=-1, keepdims=True)
    var = jnp.mean((m - mu) * (m - mu), axis=-1, keepdims=True)
    nnf = (m - mu) * lax.rsqrt(var + 1e-5) * g_ref[...] + beta_ref[...]
    out_ref[...] = node + nnf


def _tc_node(node_pad, agg0, agg1, n_W1, n_b1, n_W2, n_b2, n_g, n_beta):
    B = 512
    grid = (N_PAD // B,)
    row = lambda i: (i, 0)
    recv_row = lambda i: (i + N_PAD // B, 0)
    rep = lambda i: (0, 0)
    blk = pl.BlockSpec((B, C), row)
    rblk = pl.BlockSpec((B, C), recv_row)
    wspec = pl.BlockSpec((C, H), rep)
    vspec = pl.BlockSpec((1, C), rep)
    return pl.pallas_call(
        _node_body,
        grid=grid,
        in_specs=[blk, blk, blk, rblk, rblk,
                  wspec, wspec, wspec, vspec,
                  pl.BlockSpec((H, C), rep), vspec, vspec, vspec],
        out_specs=blk,
        out_shape=jax.ShapeDtypeStruct((N_PAD, C), jnp.float32),
    )(node_pad, agg0, agg1, agg0, agg1,
      n_W1[0:C], n_W1[C:2 * C], n_W1[2 * C:3 * C], n_b1.reshape(1, H),
      n_W2, n_b2.reshape(1, C), n_g.reshape(1, C), n_beta.reshape(1, C))


def kernel(node_emb, edge_emb, neighbour_list, e_W1, e_b1, e_W2, e_b2, e_g,
           e_beta, n_W1, n_b1, n_W2, n_b2, n_g, n_beta, r_W, r_b, s_W, s_b):
    senders = neighbour_list[0]
    receivers = neighbour_list[1]

    eo = []
    wsr = []
    aggs = []
    for half in range(NHALF):
        gather, _ = _sc_kernels(half * EPART, EPART)
        s_rows, r_rows = gather(node_emb, senders, receivers)
        eo_h, ws_h, wr_h = _tc_edge(half * (EPART // 4000), edge_emb, s_rows,
                                    r_rows, e_W1, e_b1, e_W2, e_b2, e_g,
                                    e_beta, r_W, r_b, s_W, s_b)
        eo.append(eo_h)
        wsr.append((ws_h, wr_h))
    for half in range(NHALF):
        _, scatter = _sc_kernels(half * EPART, EPART)
        aggs.append(scatter(wsr[half][0], wsr[half][1], senders, receivers))

    edge_out = jnp.concatenate(eo, axis=0)
    node_pad = jnp.pad(node_emb, ((0, N_PAD - N), (0, 0)))
    node_out = _tc_node(node_pad, aggs[0], aggs[1], n_W1, n_b1, n_W2, n_b2,
                        n_g, n_beta)[:N]
    return (node_out, edge_out)


# speedup vs baseline: 4.7090x; 1.0074x over previous
import functools

import jax
import jax.numpy as jnp
from jax import lax
from jax.experimental import pallas as pl
from jax.experimental.pallas import tpu as pltpu
from jax.experimental.pallas import tpu_sc as plsc

N = 10000
E = 320000
C = 128
H = 128

N_PAD = 10240
NW = 32
KG = 40
KS = 80
ROWS_PT = N_PAD // 16
ZROWS = ROWS_PT // 8
NHALF = 2
EPART = E // NHALF


def _pipe(nch, nbuf, start_fill, wait_fill, start_drain, wait_drain):
    assert nch >= 2 * nbuf
    for b in range(nbuf):
        start_fill(b, b)
    ngroups = nch // nbuf
    rem = nch % nbuf

    def body(k, carry):
        c = nbuf * k
        for b in range(nbuf):
            wait_fill(b)
            start_drain(c + b, b)
            if b >= 1:
                wait_drain(b - 1)
                start_fill(c + nbuf + b - 1, b - 1)
        wait_drain(nbuf - 1)
        start_fill(c + 2 * nbuf - 1, nbuf - 1)
        return carry

    lax.fori_loop(0, ngroups - 1, body, 0)
    c = nbuf * (ngroups - 1)
    for b in range(nbuf):
        wait_fill(b)
        start_drain(c + b, b)
    for j in range(rem):
        ci = nbuf * ngroups + j
        wait_drain(j)
        start_fill(ci, j)
        wait_fill(j)
        start_drain(ci, j)
    for b in range(nbuf):
        wait_drain(b)


@functools.cache
def _sc_kernels(base, e_part):
    mesh = plsc.VectorSubcoreMesh(core_axis_name="c", subcore_axis_name="s")
    epw = e_part // NW
    nchg = epw // KG
    ept = e_part // 16
    nchs = ept // KS

    nbg = 4
    @functools.partial(
        pl.kernel,
        out_type=[
            jax.ShapeDtypeStruct((e_part, C), jnp.float32),
            jax.ShapeDtypeStruct((e_part, C), jnp.float32),
        ],
        mesh=mesh,
        scratch_types=(
            [pltpu.VMEM((epw,), jnp.int32)] * 2
            + [pltpu.VMEM((KG, C), jnp.float32)] * (2 * nbg)
            + [pltpu.SemaphoreType.DMA] * (4 * nbg)
        ),
    )
    def sc_gather(node_hbm, snd_hbm, rcv_hbm, outs_hbm, outr_hbm, *scr):
        sidx, ridx = scr[0], scr[1]
        sbufs = scr[2:2 + nbg]
        rbufs = scr[2 + nbg:2 + 2 * nbg]
        sems = scr[2 + 2 * nbg:]
        gssem = sems[0:nbg]
        grsem = sems[nbg:2 * nbg]
        wssem = sems[2 * nbg:3 * nbg]
        wrsem = sems[3 * nbg:4 * nbg]

        wid = lax.axis_index("s") * 2 + lax.axis_index("c")
        lbase = wid * epw
        c0 = pltpu.async_copy(snd_hbm.at[pl.ds(base + lbase, epw)], sidx,
                              gssem[0])
        c1 = pltpu.async_copy(rcv_hbm.at[pl.ds(base + lbase, epw)], ridx,
                              grsem[0])
        c0.wait()
        c1.wait()

        def start_fill(ci, b):
            pltpu.async_copy(node_hbm.at[sidx.at[pl.ds(ci * KG, KG)]],
                             sbufs[b], gssem[b])
            pltpu.async_copy(node_hbm.at[ridx.at[pl.ds(ci * KG, KG)]],
                             rbufs[b], grsem[b])

        def wait_fill(b):
            pltpu.make_async_copy(node_hbm.at[pl.ds(0, KG)], sbufs[b],
                                  gssem[b]).wait()
            pltpu.make_async_copy(node_hbm.at[pl.ds(0, KG)], rbufs[b],
                                  grsem[b]).wait()

        def start_drain(ci, b):
            off = lbase + ci * KG
            pltpu.async_copy(sbufs[b], outs_hbm.at[pl.ds(off, KG)], wssem[b])
            pltpu.async_copy(rbufs[b], outr_hbm.at[pl.ds(off, KG)], wrsem[b])

        def wait_drain(b):
            pltpu.make_async_copy(sbufs[b], outs_hbm.at[pl.ds(0, KG)],
                                  wssem[b]).wait()
            pltpu.make_async_copy(rbufs[b], outr_hbm.at[pl.ds(0, KG)],
                                  wrsem[b]).wait()

        _pipe(nchg, nbg, start_fill, wait_fill, start_drain, wait_drain)

    nbs = 3
    @functools.partial(
        pl.kernel,
        out_type=jax.ShapeDtypeStruct((2 * N_PAD, C), jnp.float32),
        mesh=mesh,
        scratch_types=(
            [pltpu.VMEM_SHARED((N_PAD, C), jnp.float32)]
            + [pltpu.VMEM((KS,), jnp.int32)] * nbs
            + [pltpu.VMEM((KS, C), jnp.float32)] * nbs
            + [pltpu.VMEM((ZROWS, C), jnp.float32)]
            + [pltpu.SemaphoreType.DMA] * (3 * nbs)
        ),
    )
    def sc_scatter(ws_hbm, wr_hbm, snd_hbm, rcv_hbm, out_hbm, *scr):
        acc = scr[0]
        ibufs = scr[1:1 + nbs]
        dbufs = scr[1 + nbs:1 + 2 * nbs]
        zb = scr[1 + 2 * nbs]
        sems = scr[2 + 2 * nbs:]
        isem = sems[0:nbs]
        dsem = sems[nbs:2 * nbs]
        asem = sems[2 * nbs:3 * nbs]

        cid = lax.axis_index("c")
        sid = lax.axis_index("s")

        def zrow(i, carry):
            for j in range(C // 16):
                zb[i, pl.ds(j * 16, 16)] = jnp.zeros((16,), jnp.float32)
            return carry

        lax.fori_loop(0, ZROWS, zrow, 0)
        for i in range(ROWS_PT // ZROWS):
            pltpu.sync_copy(
                zb, acc.at[pl.ds(sid * ROWS_PT + i * ZROWS, ZROWS)])
        plsc.subcore_barrier()

        def process(data_hbm, idx_hbm):
            def start_fill(ci, b):
                pltpu.async_copy(
                    idx_hbm.at[pl.ds(base + sid * ept + ci * KS, KS)],
                    ibufs[b], isem[b])
                pltpu.async_copy(
                    data_hbm.at[pl.ds(sid * ept + ci * KS, KS)],
                    dbufs[b], dsem[b])

            def wait_fill(b):
                pltpu.make_async_copy(idx_hbm.at[pl.ds(0, KS)], ibufs[b],
                                      isem[b]).wait()
                pltpu.make_async_copy(data_hbm.at[pl.ds(0, KS)], dbufs[b],
                                      dsem[b]).wait()

            def start_drain(ci, b):
                pltpu.async_copy(dbufs[b], acc.at[ibufs[b]], asem[b],
                                 add=True)

            def wait_drain(b):
                pltpu.make_async_copy(dbufs[b], acc.at[pl.ds(0, KS)],
                                      asem[b]).wait()

            _pipe(nchs, nbs, start_fill, wait_fill, start_drain, wait_drain)

        @pl.when(cid == 0)
        def _():
            process(ws_hbm, snd_hbm)

        @pl.when(cid == 1)
        def _():
            process(wr_hbm, rcv_hbm)

        plsc.subcore_barrier()
        pltpu.sync_copy(
            acc.at[pl.ds(sid * ROWS_PT, ROWS_PT)],
            out_hbm.at[pl.ds(cid * N_PAD + sid * ROWS_PT, ROWS_PT)])

    return sc_gather, sc_scatter


def _edge_body(edge_ref, s_ref, r_ref, w1a_ref, w1b_ref, w1c_ref, b1_ref,
               w2_ref, b2_ref, g_ref, beta_ref, rw_ref, rb_ref, sw_ref,
               sb_ref, eout_ref, ws_ref, wr_ref):
    edge = edge_ref[...]
    h = (jnp.dot(edge, w1a_ref[...], preferred_element_type=jnp.float32)
         + jnp.dot(s_ref[...], w1b_ref[...], preferred_element_type=jnp.float32)
         + jnp.dot(r_ref[...], w1c_ref[...], preferred_element_type=jnp.float32)
         + b1_ref[...])
    h = h * jax.nn.sigmoid(h)
    m = jnp.dot(h, w2_ref[...], preferred_element_type=jnp.float32) + b2_ref[...]
    mu = jnp.mean(m, axis=-1, keepdims=True)
    var = jnp.mean((m - mu) * (m - mu), axis=-1, keepdims=True)
    nef = (m - mu) * lax.rsqrt(var + 1e-5) * g_ref[...] + beta_ref[...]
    ra = jax.nn.sigmoid(
        jnp.sum(edge * rw_ref[...], axis=-1, keepdims=True) + rb_ref[0, 0])
    sa = jax.nn.sigmoid(
        jnp.sum(edge * sw_ref[...], axis=-1, keepdims=True) + sb_ref[0, 0])
    eout_ref[...] = edge + nef
    ws_ref[...] = nef * sa
    wr_ref[...] = nef * ra


def _tc_edge(block_off, edge_emb, s_rows, r_rows, e_W1, e_b1, e_W2, e_b2,
             e_g, e_beta, r_W, r_b, s_W, s_b):
    B = 8000
    grid = (EPART // B,)
    full_row = lambda i: (i + block_off, 0)
    row = lambda i: (i, 0)
    rep = lambda i: (0, 0)
    blk = pl.BlockSpec((B, C), row)
    wspec = pl.BlockSpec((C, H), rep)
    vspec = pl.BlockSpec((1, C), rep)
    sspec = pl.BlockSpec((1, 1), rep)
    return pl.pallas_call(
        _edge_body,
        grid=grid,
        in_specs=[pl.BlockSpec((B, C), full_row), blk, blk,
                  wspec, wspec, wspec, vspec,
                  pl.BlockSpec((H, C), rep), vspec, vspec, vspec,
                  vspec, sspec, vspec, sspec],
        out_specs=[blk, blk, blk],
        out_shape=[jax.ShapeDtypeStruct((EPART, C), jnp.float32)] * 3,
    )(edge_emb, s_rows, r_rows,
      e_W1[0:C], e_W1[C:2 * C], e_W1[2 * C:3 * C], e_b1.reshape(1, H),
      e_W2, e_b2.reshape(1, C), e_g.reshape(1, C), e_beta.reshape(1, C),
      r_W.reshape(1, C), r_b.reshape(1, 1),
      s_W.reshape(1, C), s_b.reshape(1, 1))


def _node_body(node_ref, s0_ref, s1_ref, r0_ref, r1_ref, w1a_ref, w1b_ref,
               w1c_ref, b1_ref, w2_ref, b2_ref, g_ref, beta_ref, out_ref):
    node = node_ref[...]
    snt = s0_ref[...] + s1_ref[...]
    rcv = r0_ref[...] + r1_ref[...]
    h = (jnp.dot(node, w1a_ref[...], preferred_element_type=jnp.float32)
         + jnp.dot(snt, w1b_ref[...], preferred_element_type=jnp.float32)
         + jnp.dot(rcv, w1c_ref[...], preferred_element_type=jnp.float32)
         + b1_ref[...])
    h = h * jax.nn.sigmoid(h)
    m = jnp.dot(h, w2_ref[...], preferred_element_type=jnp.float32) + b2_ref[...]
    mu = jnp.mean(m, axis=-1, keepdims=True)
    var = jnp.mean((m - mu) * (m - mu), axis=-1, keepdims=True)
    nnf = (m - mu) * lax.rsqrt(var + 1e-5) * g_ref[...] + beta_ref[...]
    out_ref[...] = node + nnf


def _tc_node(node_pad, agg0, agg1, n_W1, n_b1, n_W2, n_b2, n_g, n_beta):
    B = 512
    grid = (N_PAD // B,)
    row = lambda i: (i, 0)
    recv_row = lambda i: (i + N_PAD // B, 0)
    rep = lambda i: (0, 0)
    blk = pl.BlockSpec((B, C), row)
    rblk = pl.BlockSpec((B, C), recv_row)
    wspec = pl.BlockSpec((C, H), rep)
    vspec = pl.BlockSpec((1, C), rep)
    return pl.pallas_call(
        _node_body,
        grid=grid,
        in_specs=[blk, blk, blk, rblk, rblk,
                  wspec, wspec, wspec, vspec,
                  pl.BlockSpec((H, C), rep), vspec, vspec, vspec],
        out_specs=blk,
        out_shape=jax.ShapeDtypeStruct((N_PAD, C), jnp.float32),
    )(node_pad, agg0, agg1, agg0, agg1,
      n_W1[0:C], n_W1[C:2 * C], n_W1[2 * C:3 * C], n_b1.reshape(1, H),
      n_W2, n_b2.reshape(1, C), n_g.reshape(1, C), n_beta.reshape(1, C))


def kernel(node_emb, edge_emb, neighbour_list, e_W1, e_b1, e_W2, e_b2, e_g,
           e_beta, n_W1, n_b1, n_W2, n_b2, n_g, n_beta, r_W, r_b, s_W, s_b):
    senders = neighbour_list[0]
    receivers = neighbour_list[1]

    eo = []
    wsr = []
    aggs = []
    for half in range(NHALF):
        gather, _ = _sc_kernels(half * EPART, EPART)
        s_rows, r_rows = gather(node_emb, senders, receivers)
        eo_h, ws_h, wr_h = _tc_edge(half * (EPART // 8000), edge_emb, s_rows,
                                    r_rows, e_W1, e_b1, e_W2, e_b2, e_g,
                                    e_beta, r_W, r_b, s_W, s_b)
        eo.append(eo_h)
        wsr.append((ws_h, wr_h))
    for half in range(NHALF):
        _, scatter = _sc_kernels(half * EPART, EPART)
        aggs.append(scatter(wsr[half][0], wsr[half][1], senders, receivers))

    edge_out = jnp.concatenate(eo, axis=0)
    node_pad = jnp.pad(node_emb, ((0, N_PAD - N), (0, 0)))
    node_out = _tc_node(node_pad, aggs[0], aggs[1], n_W1, n_b1, n_W2, n_b2,
                        n_g, n_beta)[:N]
    return (node_out, edge_out)


# speedup vs baseline: 5.0988x; 1.0828x over previous
import functools

import jax
import jax.numpy as jnp
from jax import lax
from jax.experimental import pallas as pl
from jax.experimental.pallas import tpu as pltpu
from jax.experimental.pallas import tpu_sc as plsc

N = 10000
E = 320000
C = 128
H = 128

N_PAD = 10240
NW = 32
KG = 40
KS = 80
ROWS_PT = N_PAD // 16
ZROWS = ROWS_PT // 8
NHALF = 2
EPART = E // NHALF


def _pipe(nch, nbuf, start_fill, wait_fill, start_drain, wait_drain):
    assert nch >= 2 * nbuf
    for b in range(nbuf):
        start_fill(b, b)
    ngroups = nch // nbuf
    rem = nch % nbuf

    def body(k, carry):
        c = nbuf * k
        for b in range(nbuf):
            wait_fill(b)
            start_drain(c + b, b)
            if b >= 1:
                wait_drain(b - 1)
                start_fill(c + nbuf + b - 1, b - 1)
        wait_drain(nbuf - 1)
        start_fill(c + 2 * nbuf - 1, nbuf - 1)
        return carry

    lax.fori_loop(0, ngroups - 1, body, 0)
    c = nbuf * (ngroups - 1)
    for b in range(nbuf):
        wait_fill(b)
        start_drain(c + b, b)
    for j in range(rem):
        ci = nbuf * ngroups + j
        wait_drain(j)
        start_fill(ci, j)
        wait_fill(j)
        start_drain(ci, j)
    for b in range(nbuf):
        wait_drain(b)


@functools.cache
def _sc_kernels(base, e_part):
    mesh = plsc.VectorSubcoreMesh(core_axis_name="c", subcore_axis_name="s")
    epw = e_part // NW
    nchg = epw // KG
    ept = e_part // 16
    nchs = ept // KS

    nbg = 4
    @functools.partial(
        pl.kernel,
        out_type=[
            jax.ShapeDtypeStruct((e_part, C), jnp.float32),
            jax.ShapeDtypeStruct((e_part, C), jnp.float32),
        ],
        mesh=mesh,
        scratch_types=(
            [pltpu.VMEM((epw,), jnp.int32)] * 2
            + [pltpu.VMEM((KG, C), jnp.float32)] * (2 * nbg)
            + [pltpu.SemaphoreType.DMA] * (4 * nbg)
        ),
    )
    def sc_gather(node_hbm, snd_hbm, rcv_hbm, outs_hbm, outr_hbm, *scr):
        sidx, ridx = scr[0], scr[1]
        sbufs = scr[2:2 + nbg]
        rbufs = scr[2 + nbg:2 + 2 * nbg]
        sems = scr[2 + 2 * nbg:]
        gssem = sems[0:nbg]
        grsem = sems[nbg:2 * nbg]
        wssem = sems[2 * nbg:3 * nbg]
        wrsem = sems[3 * nbg:4 * nbg]

        wid = lax.axis_index("s") * 2 + lax.axis_index("c")
        lbase = wid * epw
        c0 = pltpu.async_copy(snd_hbm.at[pl.ds(base + lbase, epw)], sidx,
                              gssem[0])
        c1 = pltpu.async_copy(rcv_hbm.at[pl.ds(base + lbase, epw)], ridx,
                              grsem[0])
        c0.wait()
        c1.wait()

        def start_fill(ci, b):
            pltpu.async_copy(node_hbm.at[sidx.at[pl.ds(ci * KG, KG)]],
                             sbufs[b], gssem[b])
            pltpu.async_copy(node_hbm.at[ridx.at[pl.ds(ci * KG, KG)]],
                             rbufs[b], grsem[b])

        def wait_fill(b):
            pltpu.make_async_copy(node_hbm.at[pl.ds(0, KG)], sbufs[b],
                                  gssem[b]).wait()
            pltpu.make_async_copy(node_hbm.at[pl.ds(0, KG)], rbufs[b],
                                  grsem[b]).wait()

        def start_drain(ci, b):
            off = lbase + ci * KG
            pltpu.async_copy(sbufs[b], outs_hbm.at[pl.ds(off, KG)], wssem[b])
            pltpu.async_copy(rbufs[b], outr_hbm.at[pl.ds(off, KG)], wrsem[b])

        def wait_drain(b):
            pltpu.make_async_copy(sbufs[b], outs_hbm.at[pl.ds(0, KG)],
                                  wssem[b]).wait()
            pltpu.make_async_copy(rbufs[b], outr_hbm.at[pl.ds(0, KG)],
                                  wrsem[b]).wait()

        _pipe(nchg, nbg, start_fill, wait_fill, start_drain, wait_drain)

    nbs = 3
    @functools.partial(
        pl.kernel,
        out_type=jax.ShapeDtypeStruct((2 * N_PAD, C), jnp.float32),
        mesh=mesh,
        scratch_types=(
            [pltpu.VMEM_SHARED((N_PAD, C), jnp.float32)]
            + [pltpu.VMEM((KS,), jnp.int32)] * nbs
            + [pltpu.VMEM((KS, C), jnp.float32)] * nbs
            + [pltpu.VMEM((ZROWS, C), jnp.float32)]
            + [pltpu.SemaphoreType.DMA] * (3 * nbs)
        ),
    )
    def sc_scatter(ws_hbm, wr_hbm, snd_hbm, rcv_hbm, out_hbm, *scr):
        acc = scr[0]
        ibufs = scr[1:1 + nbs]
        dbufs = scr[1 + nbs:1 + 2 * nbs]
        zb = scr[1 + 2 * nbs]
        sems = scr[2 + 2 * nbs:]
        isem = sems[0:nbs]
        dsem = sems[nbs:2 * nbs]
        asem = sems[2 * nbs:3 * nbs]

        cid = lax.axis_index("c")
        sid = lax.axis_index("s")

        def zrow(i, carry):
            for j in range(C // 16):
                zb[i, pl.ds(j * 16, 16)] = jnp.zeros((16,), jnp.float32)
            return carry

        lax.fori_loop(0, ZROWS, zrow, 0)
        for i in range(ROWS_PT // ZROWS):
            pltpu.sync_copy(
                zb, acc.at[pl.ds(sid * ROWS_PT + i * ZROWS, ZROWS)])
        plsc.subcore_barrier()

        def process(data_hbm, idx_hbm):
            def start_fill(ci, b):
                pltpu.async_copy(
                    idx_hbm.at[pl.ds(base + sid * ept + ci * KS, KS)],
                    ibufs[b], isem[b])
                pltpu.async_copy(
                    data_hbm.at[pl.ds(sid * ept + ci * KS, KS)],
                    dbufs[b], dsem[b])

            def wait_fill(b):
                pltpu.make_async_copy(idx_hbm.at[pl.ds(0, KS)], ibufs[b],
                                      isem[b]).wait()
                pltpu.make_async_copy(data_hbm.at[pl.ds(0, KS)], dbufs[b],
                                      dsem[b]).wait()

            def start_drain(ci, b):
                pltpu.async_copy(dbufs[b], acc.at[ibufs[b]], asem[b],
                                 add=True)

            def wait_drain(b):
                pltpu.make_async_copy(dbufs[b], acc.at[pl.ds(0, KS)],
                                      asem[b]).wait()

            _pipe(nchs, nbs, start_fill, wait_fill, start_drain, wait_drain)

        @pl.when(cid == 0)
        def _():
            process(ws_hbm, snd_hbm)

        @pl.when(cid == 1)
        def _():
            process(wr_hbm, rcv_hbm)

        plsc.subcore_barrier()
        pltpu.sync_copy(
            acc.at[pl.ds(sid * ROWS_PT, ROWS_PT)],
            out_hbm.at[pl.ds(cid * N_PAD + sid * ROWS_PT, ROWS_PT)])

    return sc_gather, sc_scatter


def _edge_body(prev_ref, edge_ref, s_ref, r_ref, w1a_ref, w1b_ref, w1c_ref,
               b1_ref, w2_ref, b2_ref, g_ref, beta_ref, rw_ref, rb_ref,
               sw_ref, sb_ref, eout_ref, ws_ref, wr_ref):
    del prev_ref
    edge = edge_ref[...]
    s = s_ref[...]
    r = r_ref[...]
    h = (jnp.dot(edge, w1a_ref[...], preferred_element_type=jnp.float32)
         + jnp.dot(s, w1b_ref[...], preferred_element_type=jnp.float32)
         + jnp.dot(r, w1c_ref[...], preferred_element_type=jnp.float32)
         + b1_ref[...])
    h = h * jax.nn.sigmoid(h)
    m = jnp.dot(h, w2_ref[...], preferred_element_type=jnp.float32) + b2_ref[...]
    mu = jnp.mean(m, axis=-1, keepdims=True)
    var = jnp.mean((m - mu) * (m - mu), axis=-1, keepdims=True)
    nef = (m - mu) * lax.rsqrt(var + 1e-5) * g_ref[...] + beta_ref[...]
    ra = jax.nn.sigmoid(
        jnp.sum(edge * rw_ref[...], axis=-1, keepdims=True) + rb_ref[0, 0])
    sa = jax.nn.sigmoid(
        jnp.sum(edge * sw_ref[...], axis=-1, keepdims=True) + sb_ref[0, 0])
    eout_ref[...] = edge + nef
    ws_ref[...] = nef * sa
    wr_ref[...] = nef * ra


def _tc_edge(block_off, eo_prev, edge_emb, s_rows, r_rows, e_W1, e_b1, e_W2,
             e_b2, e_g, e_beta, r_W, r_b, s_W, s_b):
    B = 8000
    grid = (EPART // B,)
    full_row = lambda i: (i + block_off, 0)
    row = lambda i: (i, 0)
    rep = lambda i: (0, 0)
    blk = pl.BlockSpec((B, C), row)
    wspec = pl.BlockSpec((C, H), rep)
    vspec = pl.BlockSpec((1, C), rep)
    sspec = pl.BlockSpec((1, 1), rep)
    donor = edge_emb if eo_prev is None else eo_prev
    aliases = {} if eo_prev is None else {0: 0}
    return pl.pallas_call(
        _edge_body,
        grid=grid,
        in_specs=[pl.BlockSpec((8, C), rep),
                  pl.BlockSpec((B, C), full_row), blk, blk,
                  wspec, wspec, wspec, vspec,
                  pl.BlockSpec((H, C), rep), vspec, vspec, vspec,
                  vspec, sspec, vspec, sspec],
        out_specs=[pl.BlockSpec((B, C), full_row), blk, blk],
        out_shape=[jax.ShapeDtypeStruct((E, C), jnp.float32)]
        + [jax.ShapeDtypeStruct((EPART, C), jnp.float32)] * 2,
        input_output_aliases=aliases,
    )(donor, edge_emb, s_rows, r_rows,
      e_W1[0:C], e_W1[C:2 * C], e_W1[2 * C:3 * C], e_b1.reshape(1, H),
      e_W2, e_b2.reshape(1, C), e_g.reshape(1, C), e_beta.reshape(1, C),
      r_W.reshape(1, C), r_b.reshape(1, 1),
      s_W.reshape(1, C), s_b.reshape(1, 1))


def _node_body(node_ref, s0_ref, s1_ref, r0_ref, r1_ref, w1a_ref, w1b_ref,
               w1c_ref, b1_ref, w2_ref, b2_ref, g_ref, beta_ref, out_ref):
    node = node_ref[...]
    snt = s0_ref[...] + s1_ref[...]
    rcv = r0_ref[...] + r1_ref[...]
    h = (jnp.dot(node, w1a_ref[...], preferred_element_type=jnp.float32)
         + jnp.dot(snt, w1b_ref[...], preferred_element_type=jnp.float32)
         + jnp.dot(rcv, w1c_ref[...], preferred_element_type=jnp.float32)
         + b1_ref[...])
    h = h * jax.nn.sigmoid(h)
    m = jnp.dot(h, w2_ref[...], preferred_element_type=jnp.float32) + b2_ref[...]
    mu = jnp.mean(m, axis=-1, keepdims=True)
    var = jnp.mean((m - mu) * (m - mu), axis=-1, keepdims=True)
    nnf = (m - mu) * lax.rsqrt(var + 1e-5) * g_ref[...] + beta_ref[...]
    out_ref[...] = node + nnf


def _tc_node(node_pad, agg0, agg1, n_W1, n_b1, n_W2, n_b2, n_g, n_beta):
    B = 512
    grid = (N_PAD // B,)
    row = lambda i: (i, 0)
    recv_row = lambda i: (i + N_PAD // B, 0)
    rep = lambda i: (0, 0)
    blk = pl.BlockSpec((B, C), row)
    rblk = pl.BlockSpec((B, C), recv_row)
    wspec = pl.BlockSpec((C, H), rep)
    vspec = pl.BlockSpec((1, C), rep)
    return pl.pallas_call(
        _node_body,
        grid=grid,
        in_specs=[blk, blk, blk, rblk, rblk,
                  wspec, wspec, wspec, vspec,
                  pl.BlockSpec((H, C), rep), vspec, vspec, vspec],
        out_specs=blk,
        out_shape=jax.ShapeDtypeStruct((N_PAD, C), jnp.float32),
    )(node_pad, agg0, agg1, agg0, agg1,
      n_W1[0:C], n_W1[C:2 * C], n_W1[2 * C:3 * C], n_b1.reshape(1, H),
      n_W2, n_b2.reshape(1, C), n_g.reshape(1, C), n_beta.reshape(1, C))


def kernel(node_emb, edge_emb, neighbour_list, e_W1, e_b1, e_W2, e_b2, e_g,
           e_beta, n_W1, n_b1, n_W2, n_b2, n_g, n_beta, r_W, r_b, s_W, s_b):
    senders = neighbour_list[0]
    receivers = neighbour_list[1]
    eo_prev = None
    wsr = []
    aggs = []
    for half in range(NHALF):
        gather, _ = _sc_kernels(half * EPART, EPART)
        s_rows, r_rows = gather(node_emb, senders, receivers)
        eo_prev, ws_h, wr_h = _tc_edge(half * (EPART // 8000), eo_prev,
                                       edge_emb, s_rows, r_rows, e_W1, e_b1,
                                       e_W2, e_b2, e_g, e_beta, r_W, r_b,
                                       s_W, s_b)
        wsr.append((ws_h, wr_h))
    for half in range(NHALF):
        _, scatter = _sc_kernels(half * EPART, EPART)
        aggs.append(scatter(wsr[half][0], wsr[half][1], senders, receivers))

    edge_out = eo_prev
    node_pad = jnp.pad(node_emb, ((0, N_PAD - N), (0, 0)))
    node_out = _tc_node(node_pad, aggs[0], aggs[1], n_W1, n_b1, n_W2, n_b2,
                        n_g, n_beta)[:N]
    return (node_out, edge_out)
